# baseline (speedup 1.0000x reference)
import jax
import jax.numpy as jnp
from jax import lax
from jax.experimental import pallas as pl
from jax.experimental.pallas import tpu as pltpu
from jax.experimental.pallas import tpu_sc as plsc

N = 10000
E = 320000
D = 128
B = 16384

NC = 2
NS = 16
NW = NC * NS
L = 16

NP = 10112
RPT = NP // NS
K2 = 120
NCHUNK = 87
EPW = NCHUNK * K2
E2 = EPW * NW
BPW = B // NW
KB = 128
NBCHUNK = BPW // KB


def _lane_splat(vec, r):
    idx = jnp.full((L, 1), 0, jnp.int32) + r
    dn = lax.GatherDimensionNumbers(
        offset_dims=(), collapsed_slice_dims=(0,), start_index_map=(0,))
    return lax.gather(vec, idx, dn, (1,),
                      mode=lax.GatherScatterMode.PROMISE_IN_BOUNDS)


def _scale_chunk(rows_ref, ew_ref):
    def grp(g, carry):
        ew_vec = ew_ref[pl.ds(g * L, L)]
        for r in range(L):
            sv = _lane_splat(ew_vec, r)
            for j in range(D // L):
                rows_ref[g * L + r, pl.ds(j * L, L)] = (
                    rows_ref[g * L + r, pl.ds(j * L, L)] * sv)
        return carry
    lax.fori_loop(0, K2 // L, grp, 0)
    tail = (K2 // L) * L
    ew_vec = ew_ref[pl.ds(tail, L)]
    for r in range(K2 - tail):
        sv = _lane_splat(ew_vec, r)
        for j in range(D // L):
            rows_ref[tail + r, pl.ds(j * L, L)] = (
                rows_ref[tail + r, pl.ds(j * L, L)] * sv)


def _scatter_body(src_h, dst_h, ew_h, w_h, p0_h, p1_h,
                  sb0, sb1, sb2, eb0, eb1, eb2,
                  r0, r1, r2, t0, t1, t2, acc,
                  sg0, sg1, sg2, ss0, ss1, ss2, si0, si1, si2,
                  sd0, sd1, sd2):
    sb = [sb0, sb1, sb2]
    eb = [eb0, eb1, eb2]
    rows = [r0, r1, r2]
    dss = [t0, t1, t2]
    sg = [sg0, sg1, sg2]
    ss = [ss0, ss1, ss2]
    si = [si0, si1, si2]
    sd = [sd0, sd1, sd2]

    c = lax.axis_index("c")
    s = lax.axis_index("s")
    wid = s * NC + c
    ebase = wid * EPW

    def zrow(i, carry):
        for j in range(D // L):
            r0[i, pl.ds(j * L, L)] = jnp.zeros((L,), jnp.float32)
        return carry
    lax.fori_loop(0, K2, zrow, 0)
    for t in range(5):
        pltpu.sync_copy(r0, acc.at[pl.ds(s * RPT + t * K2, K2)])
    pltpu.sync_copy(r0.at[pl.ds(0, RPT - 5 * K2)],
                    acc.at[pl.ds(s * RPT + 5 * K2, RPT - 5 * K2)])
    plsc.subcore_barrier()

    def fetch_se(i, b):
        off = ebase + i * K2
        pltpu.async_copy(src_h.at[pl.ds(off, K2)], sb[b], si[b])
        pltpu.async_copy(ew_h.at[pl.ds(off, K2)], eb[b].at[pl.ds(0, K2)], si[b])

    def drain_se(b):
        pltpu.make_async_copy(src_h.at[pl.ds(0, K2)], sb[b], si[b]).wait()
        pltpu.make_async_copy(ew_h.at[pl.ds(0, K2)],
                              eb[b].at[pl.ds(0, K2)], si[b]).wait()

    def fetch_dst(i, b):
        off = ebase + i * K2
        pltpu.async_copy(dst_h.at[pl.ds(off, K2)], dss[b], sd[b])

    def drain_dst(b):
        pltpu.make_async_copy(dst_h.at[pl.ds(0, K2)], dss[b], sd[b]).wait()

    for b in range(3):
        fetch_se(b, b)
    for b in range(2):
        drain_se(b)
        fetch_dst(b, b)
        pltpu.async_copy(w_h.at[sb[b]], rows[b], sg[b])

    def outer(p, carry):
        for b in range(3):
            i = 3 * p + b
            pltpu.make_async_copy(w_h.at[sb[b]], rows[b], sg[b]).wait()
            drain_dst(b)
            _scale_chunk(rows[b], eb[b])
            pltpu.async_copy(rows[b], acc.at[dss[b]], ss[b], add=True)
            @pl.when(i + 3 < NCHUNK)
            def _():
                fetch_se(i + 3, b)
            @pl.when(i + 2 < NCHUNK)
            def _():
                b2 = (b + 2) % 3
                @pl.when(i >= 1)
                def _():
                    pltpu.make_async_copy(
                        rows[b2], acc.at[dss[b2]], ss[b2]).wait()
                drain_se(b2)
                fetch_dst(i + 2, b2)
                pltpu.async_copy(w_h.at[sb[b2]], rows[b2], sg[b2])
        return carry
    lax.fori_loop(0, NCHUNK // 3, outer, 0)

    for b in range(3):
        pltpu.make_async_copy(rows[b], acc.at[dss[b]], ss[b]).wait()

    plsc.subcore_barrier()

    @pl.when(c == 0)
    def _():
        pltpu.sync_copy(acc.at[pl.ds(s * RPT, RPT)], p0_h.at[pl.ds(s * RPT, RPT)])
    @pl.when(c == 1)
    def _():
        pltpu.sync_copy(acc.at[pl.ds(s * RPT, RPT)], p1_h.at[pl.ds(s * RPT, RPT)])


def _gather_body(p0_h, p1_h, x_h, out_h, xv, rows):
    c = lax.axis_index("c")
    s = lax.axis_index("s")
    wid = s * NC + c
    for t in range(NBCHUNK):
        off = wid * BPW + t * KB
        pltpu.sync_copy(x_h.at[pl.ds(off, KB)], xv)
        pltpu.sync_copy(p0_h.at[xv], rows)
        pltpu.sync_copy(p1_h.at[xv], rows, add=True)
        pltpu.sync_copy(rows, out_h.at[pl.ds(off, KB)])


def kernel(x, edge_index, edge_weight, W):
    src = edge_index[0]
    dst = edge_index[1]
    pad = E2 - E
    src2 = jnp.concatenate([src, jnp.zeros((pad,), src.dtype)])
    dst2 = jnp.concatenate([dst, jnp.zeros((pad,), dst.dtype)])
    ew2 = jnp.concatenate([edge_weight, jnp.zeros((pad,), edge_weight.dtype)])

    mesh = plsc.VectorSubcoreMesh(core_axis_name="c", subcore_axis_name="s")

    scatter = pl.kernel(
        _scatter_body,
        mesh=mesh,
        out_type=[
            jax.ShapeDtypeStruct((NP, D), jnp.float32),
            jax.ShapeDtypeStruct((NP, D), jnp.float32),
        ],
        scratch_types=(
            [pltpu.VMEM((K2,), jnp.int32) for _ in range(3)]
            + [pltpu.VMEM((8 * L,), jnp.float32) for _ in range(3)]
            + [pltpu.VMEM((K2, D), jnp.float32) for _ in range(3)]
            + [pltpu.VMEM((K2,), jnp.int32) for _ in range(3)]
            + [pltpu.VMEM_SHARED((NP, D), jnp.float32)]
            + [pltpu.SemaphoreType.DMA for _ in range(12)]
        ),
    )
    p0, p1 = scatter(src2, dst2, ew2, W)

    gather = pl.kernel(
        _gather_body,
        mesh=mesh,
        out_type=jax.ShapeDtypeStruct((B, D), jnp.float32),
        scratch_types=[
            pltpu.VMEM((KB,), jnp.int32),
            pltpu.VMEM((KB, D), jnp.float32),
        ],
    )
    return gather(p0, p1, x)


# speedup vs baseline: 2.5510x; 2.5510x over previous
import jax
import jax.numpy as jnp
from jax import lax
from jax.experimental import pallas as pl
from jax.experimental.pallas import tpu as pltpu
from jax.experimental.pallas import tpu_sc as plsc

N = 10000
E = 320000
D = 128
B = 16384

NC = 2
NS = 16
NW = NC * NS
L = 16

NP = 10112
RPT = NP // NS
K2 = 112
NCHUNK = 90
EPW = NCHUNK * K2
E2 = EPW * NW
BPW = B // NW
KB = 128
NBCHUNK = BPW // KB


def _lane_splat(vec, r):
    idx = jnp.full((L, 1), 0, jnp.int32) + r
    dn = lax.GatherDimensionNumbers(
        offset_dims=(), collapsed_slice_dims=(0,), start_index_map=(0,))
    return lax.gather(vec, idx, dn, (1,),
                      mode=lax.GatherScatterMode.PROMISE_IN_BOUNDS)


def _scale_chunk(rows_ref, ew_ref, ebase):
    def grp(g, carry):
        ew_vec = ew_ref[pl.ds(ebase + g * L, L)]
        for r in range(L):
            sv = _lane_splat(ew_vec, r)
            for j in range(D // L):
                rows_ref[g * L + r, pl.ds(j * L, L)] = (
                    rows_ref[g * L + r, pl.ds(j * L, L)] * sv)
        return carry
    lax.fori_loop(0, K2 // L, grp, 0)


def _scatter_body(src_h, dst_h, ew_h, w_h, p0_h, p1_h,
                  srcv, ewv, r0, r1, t0, t1, acc,
                  sg0, sg1, sd0, sd1):
    rows = [r0, r1]
    dss = [t0, t1]
    sg = [sg0, sg1]
    sd = [sd0, sd1]

    c = lax.axis_index("c")
    s = lax.axis_index("s")
    wid = s * NC + c
    ebase = wid * EPW

    def zrow(i, carry):
        for j in range(D // L):
            r0[i, pl.ds(j * L, L)] = jnp.zeros((L,), jnp.float32)
        return carry
    lax.fori_loop(0, K2, zrow, 0)
    for t in range(5):
        pltpu.sync_copy(r0, acc.at[pl.ds(s * RPT + t * K2, K2)])
    pltpu.sync_copy(r0.at[pl.ds(0, RPT - 5 * K2)],
                    acc.at[pl.ds(s * RPT + 5 * K2, RPT - 5 * K2)])

    pltpu.sync_copy(src_h.at[pl.ds(ebase, EPW)], srcv)
    pltpu.sync_copy(ew_h.at[pl.ds(ebase, EPW)], ewv)
    plsc.subcore_barrier()

    def fetch_dst(i, b):
        pltpu.async_copy(dst_h.at[pl.ds(ebase + i * K2, K2)], dss[b], sd[b])

    def start_gather(i, b):
        pltpu.async_copy(w_h.at[srcv.at[pl.ds(i * K2, K2)]], rows[b], sg[b])

    for b in range(2):
        fetch_dst(b, b)
        start_gather(b, b)

    def outer(p, carry):
        for b in range(2):
            i = 2 * p + b
            pltpu.make_async_copy(
                w_h.at[srcv.at[pl.ds(0, K2)]], rows[b], sg[b]).wait()
            pltpu.make_async_copy(
                dst_h.at[pl.ds(0, K2)], dss[b], sd[b]).wait()
            _scale_chunk(rows[b], ewv, i * K2)
            pltpu.sync_copy(rows[b], acc.at[dss[b]], add=True)
            @pl.when(i + 2 < NCHUNK)
            def _():
                fetch_dst(i + 2, b)
                start_gather(i + 2, b)
        return carry
    lax.fori_loop(0, NCHUNK // 2, outer, 0)

    plsc.subcore_barrier()

    @pl.when(c == 0)
    def _():
        pltpu.sync_copy(acc.at[pl.ds(s * RPT, RPT)], p0_h.at[pl.ds(s * RPT, RPT)])
    @pl.when(c == 1)
    def _():
        pltpu.sync_copy(acc.at[pl.ds(s * RPT, RPT)], p1_h.at[pl.ds(s * RPT, RPT)])


def _gather_body(p0_h, p1_h, x_h, out_h, xv, rows):
    c = lax.axis_index("c")
    s = lax.axis_index("s")
    wid = s * NC + c
    for t in range(NBCHUNK):
        off = wid * BPW + t * KB
        pltpu.sync_copy(x_h.at[pl.ds(off, KB)], xv)
        pltpu.sync_copy(p0_h.at[xv], rows)
        pltpu.sync_copy(p1_h.at[xv], rows, add=True)
        pltpu.sync_copy(rows, out_h.at[pl.ds(off, KB)])


def kernel(x, edge_index, edge_weight, W):
    src = edge_index[0]
    dst = edge_index[1]
    pad = E2 - E
    src2 = jnp.concatenate([src, jnp.zeros((pad,), src.dtype)])
    dst2 = jnp.concatenate([dst, jnp.zeros((pad,), dst.dtype)])
    ew2 = jnp.concatenate([edge_weight, jnp.zeros((pad,), edge_weight.dtype)])

    mesh = plsc.VectorSubcoreMesh(core_axis_name="c", subcore_axis_name="s")

    scatter = pl.kernel(
        _scatter_body,
        mesh=mesh,
        out_type=[
            jax.ShapeDtypeStruct((NP, D), jnp.float32),
            jax.ShapeDtypeStruct((NP, D), jnp.float32),
        ],
        scratch_types=(
            [pltpu.VMEM((EPW,), jnp.int32),
             pltpu.VMEM((EPW,), jnp.float32)]
            + [pltpu.VMEM((K2, D), jnp.float32) for _ in range(2)]
            + [pltpu.VMEM((K2,), jnp.int32) for _ in range(2)]
            + [pltpu.VMEM_SHARED((NP, D), jnp.float32)]
            + [pltpu.SemaphoreType.DMA for _ in range(4)]
        ),
    )
    p0, p1 = scatter(src2, dst2, ew2, W)

    gather = pl.kernel(
        _gather_body,
        mesh=mesh,
        out_type=jax.ShapeDtypeStruct((B, D), jnp.float32),
        scratch_types=[
            pltpu.VMEM((KB,), jnp.int32),
            pltpu.VMEM((KB, D), jnp.float32),
        ],
    )
    return gather(p0, p1, x)


# speedup vs baseline: 2.8886x; 1.1324x over previous
import jax
import jax.numpy as jnp
from jax import lax
from jax.experimental import pallas as pl
from jax.experimental.pallas import tpu as pltpu
from jax.experimental.pallas import tpu_sc as plsc

N = 10000
E = 320000
D = 128
B = 16384

NC = 2
NS = 16
NW = NC * NS
L = 16

NP = 10112
RPT = NP // NS
K2 = 112
EPP = 180 * K2
NCH0 = 118
NCH1 = 62
EPW = NCH0 * K2
E2 = EPP * NS
BPW = B // NW
KB = 128
NBCHUNK = BPW // KB


def _lane_splat(vec, r):
    idx = jnp.full((L, 1), 0, jnp.int32) + r
    dn = lax.GatherDimensionNumbers(
        offset_dims=(), collapsed_slice_dims=(0,), start_index_map=(0,))
    return lax.gather(vec, idx, dn, (1,),
                      mode=lax.GatherScatterMode.PROMISE_IN_BOUNDS)


def _scale_chunk(rows_ref, ew_ref, ebase):
    def grp(g, carry):
        ew_vec = ew_ref[pl.ds(ebase + g * L, L)]
        for r in range(L):
            sv = _lane_splat(ew_vec, r)
            for j in range(D // L):
                rows_ref[g * L + r, pl.ds(j * L, L)] = (
                    rows_ref[g * L + r, pl.ds(j * L, L)] * sv)
        return carry
    lax.fori_loop(0, K2 // L, grp, 0)


def _scatter_body(src_h, dst_h, ew_h, w_h, p0_h, p1_h,
                  srcv, e0, e1, r0, r1, t0, t1, acc,
                  sg0, sg1, sd0, sd1):
    rows = [r0, r1]
    dss = [t0, t1]
    ewr = [e0, e1]
    sg = [sg0, sg1]
    sd = [sd0, sd1]

    c = lax.axis_index("c")
    s = lax.axis_index("s")
    ebase = s * EPP + c * (NCH0 * K2)
    nch = NCH0 - (NCH0 - NCH1) * c
    myepw = nch * K2

    def zrow(i, carry):
        for j in range(D // L):
            r0[i, pl.ds(j * L, L)] = jnp.zeros((L,), jnp.float32)
        return carry
    lax.fori_loop(0, K2, zrow, 0)
    for t in range(5):
        pltpu.sync_copy(r0, acc.at[pl.ds(s * RPT + t * K2, K2)])
    pltpu.sync_copy(r0.at[pl.ds(0, RPT - 5 * K2)],
                    acc.at[pl.ds(s * RPT + 5 * K2, RPT - 5 * K2)])

    pltpu.sync_copy(src_h.at[pl.ds(ebase, EPW)], srcv)
    plsc.subcore_barrier()

    def fetch_dst(i, b):
        pltpu.async_copy(dst_h.at[pl.ds(ebase + i * K2, K2)], dss[b], sd[b])
        pltpu.async_copy(ew_h.at[pl.ds(ebase + i * K2, K2)], ewr[b], sd[b])

    def start_gather(i, b):
        pltpu.async_copy(w_h.at[srcv.at[pl.ds(i * K2, K2)]], rows[b], sg[b])

    for b in range(2):
        fetch_dst(b, b)
        start_gather(b, b)

    def outer(p, carry):
        for b in range(2):
            i = 2 * p + b
            pltpu.make_async_copy(
                w_h.at[srcv.at[pl.ds(0, K2)]], rows[b], sg[b]).wait()
            pltpu.make_async_copy(
                dst_h.at[pl.ds(0, K2)], dss[b], sd[b]).wait()
            pltpu.make_async_copy(
                ew_h.at[pl.ds(0, K2)], ewr[b], sd[b]).wait()
            _scale_chunk(rows[b], ewr[b], 0)
            pltpu.sync_copy(rows[b], acc.at[dss[b]], add=True)
            @pl.when(i + 2 < nch)
            def _():
                fetch_dst(i + 2, b)
                start_gather(i + 2, b)
        return carry
    lax.fori_loop(0, nch // 2, outer, 0)

    plsc.subcore_barrier()

    @pl.when(c == 0)
    def _():
        pltpu.sync_copy(acc.at[pl.ds(s * RPT, RPT)], p0_h.at[pl.ds(s * RPT, RPT)])
    @pl.when(c == 1)
    def _():
        pltpu.sync_copy(acc.at[pl.ds(s * RPT, RPT)], p1_h.at[pl.ds(s * RPT, RPT)])


def _gather_body(p0_h, p1_h, x_h, out_h, xv, rows):
    c = lax.axis_index("c")
    s = lax.axis_index("s")
    wid = s * NC + c
    for t in range(NBCHUNK):
        off = wid * BPW + t * KB
        pltpu.sync_copy(x_h.at[pl.ds(off, KB)], xv)
        pltpu.sync_copy(p0_h.at[xv], rows)
        pltpu.sync_copy(p1_h.at[xv], rows, add=True)
        pltpu.sync_copy(rows, out_h.at[pl.ds(off, KB)])


def kernel(x, edge_index, edge_weight, W):
    src = edge_index[0]
    dst = edge_index[1]
    pad = E2 - E
    src2 = jnp.concatenate([src, jnp.zeros((pad,), src.dtype)])
    dst2 = jnp.concatenate([dst, jnp.zeros((pad,), dst.dtype)])
    ew2 = jnp.concatenate([edge_weight, jnp.zeros((pad,), edge_weight.dtype)])

    mesh = plsc.VectorSubcoreMesh(core_axis_name="c", subcore_axis_name="s")

    scatter = pl.kernel(
        _scatter_body,
        mesh=mesh,
        out_type=[
            jax.ShapeDtypeStruct((NP, D), jnp.float32),
            jax.ShapeDtypeStruct((NP, D), jnp.float32),
        ],
        scratch_types=(
            [pltpu.VMEM((EPW,), jnp.int32)]
            + [pltpu.VMEM((K2,), jnp.float32) for _ in range(2)]
            + [pltpu.VMEM((K2, D), jnp.float32) for _ in range(2)]
            + [pltpu.VMEM((K2,), jnp.int32) for _ in range(2)]
            + [pltpu.VMEM_SHARED((NP, D), jnp.float32)]
            + [pltpu.SemaphoreType.DMA for _ in range(4)]
        ),
    )
    p0, p1 = scatter(src2, dst2, ew2, W)

    gather = pl.kernel(
        _gather_body,
        mesh=mesh,
        out_type=jax.ShapeDtypeStruct((B, D), jnp.float32),
        scratch_types=[
            pltpu.VMEM((KB,), jnp.int32),
            pltpu.VMEM((KB, D), jnp.float32),
        ],
    )
    return gather(p0, p1, x)


# speedup vs baseline: 2.9914x; 1.0356x over previous
import jax
import jax.numpy as jnp
from jax import lax
from jax.experimental import pallas as pl
from jax.experimental.pallas import tpu as pltpu
from jax.experimental.pallas import tpu_sc as plsc

N = 10000
E = 320000
D = 128
B = 16384

NC = 2
NS = 16
NW = NC * NS
L = 16

NP = 10112
RPT = NP // NS
K2 = 112
EPP = 180 * K2
NCH0 = 126
NCH1 = 54
EPW = NCH0 * K2
E2 = EPP * NS
BPW = B // NW
KB = 128
NBCHUNK = BPW // KB


def _lane_splat(vec, r):
    idx = jnp.full((L, 1), 0, jnp.int32) + r
    dn = lax.GatherDimensionNumbers(
        offset_dims=(), collapsed_slice_dims=(0,), start_index_map=(0,))
    return lax.gather(vec, idx, dn, (1,),
                      mode=lax.GatherScatterMode.PROMISE_IN_BOUNDS)


def _scale_chunk(rows_ref, ew_ref, ebase):
    def grp(g, carry):
        ew_vec = ew_ref[pl.ds(ebase + g * L, L)]
        for r in range(L):
            sv = _lane_splat(ew_vec, r)
            for j in range(D // L):
                rows_ref[g * L + r, pl.ds(j * L, L)] = (
                    rows_ref[g * L + r, pl.ds(j * L, L)] * sv)
        return carry
    lax.fori_loop(0, K2 // L, grp, 0)


def _scatter_body(src_h, dst_h, ew_h, w_h, p0_h, p1_h,
                  srcv, e0, e1, r0, r1, t0, t1, acc,
                  sg0, sg1, sd0, sd1):
    rows = [r0, r1]
    dss = [t0, t1]
    ewr = [e0, e1]
    sg = [sg0, sg1]
    sd = [sd0, sd1]

    c = lax.axis_index("c")
    s = lax.axis_index("s")
    ebase = s * EPP + c * (NCH0 * K2)
    nch = NCH0 - (NCH0 - NCH1) * c
    myepw = nch * K2

    def zrow(i, carry):
        for j in range(D // L):
            r0[i, pl.ds(j * L, L)] = jnp.zeros((L,), jnp.float32)
        return carry
    lax.fori_loop(0, K2, zrow, 0)
    for t in range(5):
        pltpu.sync_copy(r0, acc.at[pl.ds(s * RPT + t * K2, K2)])
    pltpu.sync_copy(r0.at[pl.ds(0, RPT - 5 * K2)],
                    acc.at[pl.ds(s * RPT + 5 * K2, RPT - 5 * K2)])

    pltpu.sync_copy(src_h.at[pl.ds(ebase, EPW)], srcv)
    plsc.subcore_barrier()

    def fetch_dst(i, b):
        pltpu.async_copy(dst_h.at[pl.ds(ebase + i * K2, K2)], dss[b], sd[b])
        pltpu.async_copy(ew_h.at[pl.ds(ebase + i * K2, K2)], ewr[b], sd[b])

    def start_gather(i, b):
        pltpu.async_copy(w_h.at[srcv.at[pl.ds(i * K2, K2)]], rows[b], sg[b])

    for b in range(2):
        fetch_dst(b, b)
        start_gather(b, b)

    def outer(p, carry):
        for b in range(2):
            i = 2 * p + b
            pltpu.make_async_copy(
                w_h.at[srcv.at[pl.ds(0, K2)]], rows[b], sg[b]).wait()
            pltpu.make_async_copy(
                dst_h.at[pl.ds(0, K2)], dss[b], sd[b]).wait()
            pltpu.make_async_copy(
                ew_h.at[pl.ds(0, K2)], ewr[b], sd[b]).wait()
            _scale_chunk(rows[b], ewr[b], 0)
            pltpu.sync_copy(rows[b], acc.at[dss[b]], add=True)
            @pl.when(i + 2 < nch)
            def _():
                fetch_dst(i + 2, b)
                start_gather(i + 2, b)
        return carry
    lax.fori_loop(0, nch // 2, outer, 0)

    plsc.subcore_barrier()

    @pl.when(c == 0)
    def _():
        pltpu.sync_copy(acc.at[pl.ds(s * RPT, RPT)], p0_h.at[pl.ds(s * RPT, RPT)])
    @pl.when(c == 1)
    def _():
        pltpu.sync_copy(acc.at[pl.ds(s * RPT, RPT)], p1_h.at[pl.ds(s * RPT, RPT)])


def _gather_body(p0_h, p1_h, x_h, out_h, xv, rows):
    c = lax.axis_index("c")
    s = lax.axis_index("s")
    wid = s * NC + c
    for t in range(NBCHUNK):
        off = wid * BPW + t * KB
        pltpu.sync_copy(x_h.at[pl.ds(off, KB)], xv)
        pltpu.sync_copy(p0_h.at[xv], rows)
        pltpu.sync_copy(p1_h.at[xv], rows, add=True)
        pltpu.sync_copy(rows, out_h.at[pl.ds(off, KB)])


def kernel(x, edge_index, edge_weight, W):
    src = edge_index[0]
    dst = edge_index[1]
    pad = E2 - E
    src2 = jnp.concatenate([src, jnp.zeros((pad,), src.dtype)])
    dst2 = jnp.concatenate([dst, jnp.zeros((pad,), dst.dtype)])
    ew2 = jnp.concatenate([edge_weight, jnp.zeros((pad,), edge_weight.dtype)])

    mesh = plsc.VectorSubcoreMesh(core_axis_name="c", subcore_axis_name="s")

    scatter = pl.kernel(
        _scatter_body,
        mesh=mesh,
        out_type=[
            jax.ShapeDtypeStruct((NP, D), jnp.float32),
            jax.ShapeDtypeStruct((NP, D), jnp.float32),
        ],
        scratch_types=(
            [pltpu.VMEM((EPW,), jnp.int32)]
            + [pltpu.VMEM((K2,), jnp.float32) for _ in range(2)]
            + [pltpu.VMEM((K2, D), jnp.float32) for _ in range(2)]
            + [pltpu.VMEM((K2,), jnp.int32) for _ in range(2)]
            + [pltpu.VMEM_SHARED((NP, D), jnp.float32)]
            + [pltpu.SemaphoreType.DMA for _ in range(4)]
        ),
    )
    p0, p1 = scatter(src2, dst2, ew2, W)

    gather = pl.kernel(
        _gather_body,
        mesh=mesh,
        out_type=jax.ShapeDtypeStruct((B, D), jnp.float32),
        scratch_types=[
            pltpu.VMEM((KB,), jnp.int32),
            pltpu.VMEM((KB, D), jnp.float32),
        ],
    )
    return gather(p0, p1, x)


# speedup vs baseline: 3.0470x; 1.0186x over previous
import jax
import jax.numpy as jnp
from jax import lax
from jax.experimental import pallas as pl
from jax.experimental.pallas import tpu as pltpu
from jax.experimental.pallas import tpu_sc as plsc

N = 10000
E = 320000
D = 128
B = 16384

NC = 2
NS = 16
NW = NC * NS
L = 16

NP = 10112
RPT = NP // NS
K2 = 112
EPP = 180 * K2
NCH0 = 126
NCH1 = 54
EPW = NCH0 * K2
E2 = EPP * NS
BPW = B // NW
KB = 128
NBCHUNK = BPW // KB


def _lane_splat(vec, r):
    idx = jnp.full((L, 1), 0, jnp.int32) + r
    dn = lax.GatherDimensionNumbers(
        offset_dims=(), collapsed_slice_dims=(0,), start_index_map=(0,))
    return lax.gather(vec, idx, dn, (1,),
                      mode=lax.GatherScatterMode.PROMISE_IN_BOUNDS)


def _scale_chunk(rows_ref, ew_ref, ebase):
    def grp(g, carry):
        ew_vec = ew_ref[pl.ds(ebase + g * L, L)]
        for r in range(L):
            sv = _lane_splat(ew_vec, r)
            for j in range(D // L):
                rows_ref[g * L + r, pl.ds(j * L, L)] = (
                    rows_ref[g * L + r, pl.ds(j * L, L)] * sv)
        return carry
    lax.fori_loop(0, K2 // L, grp, 0)


def _scatter_body(src_h, dst_h, ew_h, w_h, p0_h, p1_h,
                  srcv, e0, e1, r0, r1, t0, t1, acc,
                  sg0, sg1, sd0, sd1):
    rows = [r0, r1]
    dss = [t0, t1]
    ewr = [e0, e1]
    sg = [sg0, sg1]
    sd = [sd0, sd1]

    c = lax.axis_index("c")
    s = lax.axis_index("s")
    ebase = s * EPP + c * (NCH0 * K2)
    nch = NCH0 - (NCH0 - NCH1) * c
    myepw = nch * K2

    def zrow(i, carry):
        for j in range(D // L):
            r0[i, pl.ds(j * L, L)] = jnp.zeros((L,), jnp.float32)
        return carry
    lax.fori_loop(0, K2, zrow, 0)
    for t in range(5):
        pltpu.sync_copy(r0, acc.at[pl.ds(s * RPT + t * K2, K2)])
    pltpu.sync_copy(r0.at[pl.ds(0, RPT - 5 * K2)],
                    acc.at[pl.ds(s * RPT + 5 * K2, RPT - 5 * K2)])

    pltpu.sync_copy(src_h.at[pl.ds(ebase, EPW)], srcv)
    plsc.subcore_barrier()

    def fetch_dst(i, b):
        pltpu.async_copy(dst_h.at[pl.ds(ebase + i * K2, K2)], dss[b], sd[b])
        pltpu.async_copy(ew_h.at[pl.ds(ebase + i * K2, K2)], ewr[b], sd[b])

    def start_gather(i, b):
        pltpu.async_copy(w_h.at[srcv.at[pl.ds(i * K2, K2)]], rows[b], sg[b])

    for b in range(2):
        fetch_dst(b, b)
        start_gather(b, b)

    def outer(p, carry):
        for b in range(2):
            i = 2 * p + b
            pltpu.make_async_copy(
                w_h.at[srcv.at[pl.ds(0, K2)]], rows[b], sg[b]).wait()
            pltpu.make_async_copy(
                dst_h.at[pl.ds(0, K2)], dss[b], sd[b]).wait()
            pltpu.make_async_copy(
                ew_h.at[pl.ds(0, K2)], ewr[b], sd[b]).wait()
            _scale_chunk(rows[b], ewr[b], 0)
            pltpu.sync_copy(rows[b], acc.at[dss[b]], add=True)
            @pl.when(i + 2 < nch)
            def _():
                fetch_dst(i + 2, b)
                start_gather(i + 2, b)
        return carry
    lax.fori_loop(0, nch // 2, outer, 0)

    plsc.subcore_barrier()

    @pl.when(c == 0)
    def _():
        pltpu.sync_copy(acc.at[pl.ds(s * RPT, RPT)], p0_h.at[pl.ds(s * RPT, RPT)])
    @pl.when(c == 1)
    def _():
        pltpu.sync_copy(acc.at[pl.ds(s * RPT, RPT)], p1_h.at[pl.ds(s * RPT, RPT)])


def _gather_body(p0_h, p1_h, x_h, out_h, xv,
                 r0, r1, r2, r3, g0, g1, g2, g3,
                 a0, a1, a2, a3, t0, t1, t2, t3):
    rows = [r0, r1, r2, r3]
    sgm = [g0, g1, g2, g3]
    sam = [a0, a1, a2, a3]
    stm = [t0, t1, t2, t3]
    c = lax.axis_index("c")
    s = lax.axis_index("s")
    wid = s * NC + c
    base = wid * BPW
    pltpu.sync_copy(x_h.at[pl.ds(base, BPW)], xv)
    for t in range(NBCHUNK):
        pltpu.async_copy(p0_h.at[xv.at[pl.ds(t * KB, KB)]], rows[t], sgm[t])
    for t in range(NBCHUNK):
        pltpu.make_async_copy(
            p0_h.at[xv.at[pl.ds(t * KB, KB)]], rows[t], sgm[t]).wait()
        pltpu.async_copy(p1_h.at[xv.at[pl.ds(t * KB, KB)]], rows[t],
                         sam[t], add=True)
        pltpu.make_async_copy(
            p1_h.at[xv.at[pl.ds(t * KB, KB)]], rows[t], sam[t]).wait()
        pltpu.async_copy(rows[t], out_h.at[pl.ds(base + t * KB, KB)], stm[t])
    for t in range(NBCHUNK):
        pltpu.make_async_copy(
            rows[t], out_h.at[pl.ds(base + t * KB, KB)], stm[t]).wait()


def kernel(x, edge_index, edge_weight, W):
    src = edge_index[0]
    dst = edge_index[1]
    pad = E2 - E
    src2 = jnp.concatenate([src, jnp.zeros((pad,), src.dtype)])
    dst2 = jnp.concatenate([dst, jnp.zeros((pad,), dst.dtype)])
    ew2 = jnp.concatenate([edge_weight, jnp.zeros((pad,), edge_weight.dtype)])

    mesh = plsc.VectorSubcoreMesh(core_axis_name="c", subcore_axis_name="s")

    scatter = pl.kernel(
        _scatter_body,
        mesh=mesh,
        out_type=[
            jax.ShapeDtypeStruct((NP, D), jnp.float32),
            jax.ShapeDtypeStruct((NP, D), jnp.float32),
        ],
        scratch_types=(
            [pltpu.VMEM((EPW,), jnp.int32)]
            + [pltpu.VMEM((K2,), jnp.float32) for _ in range(2)]
            + [pltpu.VMEM((K2, D), jnp.float32) for _ in range(2)]
            + [pltpu.VMEM((K2,), jnp.int32) for _ in range(2)]
            + [pltpu.VMEM_SHARED((NP, D), jnp.float32)]
            + [pltpu.SemaphoreType.DMA for _ in range(4)]
        ),
    )
    p0, p1 = scatter(src2, dst2, ew2, W)

    gather = pl.kernel(
        _gather_body,
        mesh=mesh,
        out_type=jax.ShapeDtypeStruct((B, D), jnp.float32),
        scratch_types=(
            [pltpu.VMEM((BPW,), jnp.int32)]
            + [pltpu.VMEM((KB, D), jnp.float32) for _ in range(NBCHUNK)]
            + [pltpu.SemaphoreType.DMA for _ in range(3 * NBCHUNK)]
        ),
    )
    return gather(p0, p1, x)


# speedup vs baseline: 3.0770x; 1.0099x over previous
import jax
import jax.numpy as jnp
from jax import lax
from jax.experimental import pallas as pl
from jax.experimental.pallas import tpu as pltpu
from jax.experimental.pallas import tpu_sc as plsc

N = 10000
E = 320000
D = 128
B = 16384

NC = 2
NS = 16
NW = NC * NS
L = 16

NP = 10112
RPT = NP // NS
K2 = 112
EPP = 180 * K2
NCH0 = 128
NCH1 = 52
EPW = NCH0 * K2
E2 = EPP * NS
BPW = B // NW
KB = 128
NBCHUNK = BPW // KB


def _lane_splat(vec, r):
    idx = jnp.full((L, 1), 0, jnp.int32) + r
    dn = lax.GatherDimensionNumbers(
        offset_dims=(), collapsed_slice_dims=(0,), start_index_map=(0,))
    return lax.gather(vec, idx, dn, (1,),
                      mode=lax.GatherScatterMode.PROMISE_IN_BOUNDS)


def _scale_chunk(rows_ref, ew_ref, ebase):
    def grp(g, carry):
        ew_vec = ew_ref[pl.ds(ebase + g * L, L)]
        for r in range(L):
            sv = _lane_splat(ew_vec, r)
            for j in range(D // L):
                rows_ref[g * L + r, pl.ds(j * L, L)] = (
                    rows_ref[g * L + r, pl.ds(j * L, L)] * sv)
        return carry
    lax.fori_loop(0, K2 // L, grp, 0)


def _scatter_body(src_h, dst_h, ew_h, w_h, p0_h, p1_h,
                  srcv, e0, e1, r0, r1, t0, t1, acc,
                  sg0, sg1, sd0, sd1):
    rows = [r0, r1]
    dss = [t0, t1]
    ewr = [e0, e1]
    sg = [sg0, sg1]
    sd = [sd0, sd1]

    c = lax.axis_index("c")
    s = lax.axis_index("s")
    ebase = s * EPP + c * (NCH0 * K2)
    nch = NCH0 - (NCH0 - NCH1) * c
    myepw = nch * K2

    def zrow(i, carry):
        for j in range(D // L):
            r0[i, pl.ds(j * L, L)] = jnp.zeros((L,), jnp.float32)
        return carry
    lax.fori_loop(0, K2, zrow, 0)
    for t in range(5):
        pltpu.sync_copy(r0, acc.at[pl.ds(s * RPT + t * K2, K2)])
    pltpu.sync_copy(r0.at[pl.ds(0, RPT - 5 * K2)],
                    acc.at[pl.ds(s * RPT + 5 * K2, RPT - 5 * K2)])

    pltpu.sync_copy(src_h.at[pl.ds(ebase, EPW)], srcv)
    plsc.subcore_barrier()

    def fetch_dst(i, b):
        pltpu.async_copy(dst_h.at[pl.ds(ebase + i * K2, K2)], dss[b], sd[b])
        pltpu.async_copy(ew_h.at[pl.ds(ebase + i * K2, K2)], ewr[b], sd[b])

    def start_gather(i, b):
        pltpu.async_copy(w_h.at[srcv.at[pl.ds(i * K2, K2)]], rows[b], sg[b])

    for b in range(2):
        fetch_dst(b, b)
        start_gather(b, b)

    def outer(p, carry):
        for b in range(2):
            i = 2 * p + b
            pltpu.make_async_copy(
                w_h.at[srcv.at[pl.ds(0, K2)]], rows[b], sg[b]).wait()
            pltpu.make_async_copy(
                dst_h.at[pl.ds(0, K2)], dss[b], sd[b]).wait()
            pltpu.make_async_copy(
                ew_h.at[pl.ds(0, K2)], ewr[b], sd[b]).wait()
            _scale_chunk(rows[b], ewr[b], 0)
            pltpu.sync_copy(rows[b], acc.at[dss[b]], add=True)
            @pl.when(i + 2 < nch)
            def _():
                fetch_dst(i + 2, b)
                start_gather(i + 2, b)
        return carry
    lax.fori_loop(0, nch // 2, outer, 0)

    plsc.subcore_barrier()

    @pl.when(c == 0)
    def _():
        pltpu.sync_copy(acc.at[pl.ds(s * RPT, RPT)], p0_h.at[pl.ds(s * RPT, RPT)])
    @pl.when(c == 1)
    def _():
        pltpu.sync_copy(acc.at[pl.ds(s * RPT, RPT)], p1_h.at[pl.ds(s * RPT, RPT)])


def _gather_body(p0_h, p1_h, x_h, out_h, xv,
                 r0, r1, r2, r3, g0, g1, g2, g3,
                 a0, a1, a2, a3, t0, t1, t2, t3):
    rows = [r0, r1, r2, r3]
    sgm = [g0, g1, g2, g3]
    sam = [a0, a1, a2, a3]
    stm = [t0, t1, t2, t3]
    c = lax.axis_index("c")
    s = lax.axis_index("s")
    wid = s * NC + c
    base = wid * BPW
    pltpu.sync_copy(x_h.at[pl.ds(base, BPW)], xv)
    for t in range(NBCHUNK):
        pltpu.async_copy(p0_h.at[xv.at[pl.ds(t * KB, KB)]], rows[t], sgm[t])
    for t in range(NBCHUNK):
        pltpu.make_async_copy(
            p0_h.at[xv.at[pl.ds(t * KB, KB)]], rows[t], sgm[t]).wait()
        pltpu.async_copy(p1_h.at[xv.at[pl.ds(t * KB, KB)]], rows[t],
                         sam[t], add=True)
        pltpu.make_async_copy(
            p1_h.at[xv.at[pl.ds(t * KB, KB)]], rows[t], sam[t]).wait()
        pltpu.async_copy(rows[t], out_h.at[pl.ds(base + t * KB, KB)], stm[t])
    for t in range(NBCHUNK):
        pltpu.make_async_copy(
            rows[t], out_h.at[pl.ds(base + t * KB, KB)], stm[t]).wait()


def kernel(x, edge_index, edge_weight, W):
    src = edge_index[0]
    dst = edge_index[1]
    pad = E2 - E
    src2 = jnp.concatenate([src, jnp.zeros((pad,), src.dtype)])
    dst2 = jnp.concatenate([dst, jnp.zeros((pad,), dst.dtype)])
    ew2 = jnp.concatenate([edge_weight, jnp.zeros((pad,), edge_weight.dtype)])

    mesh = plsc.VectorSubcoreMesh(core_axis_name="c", subcore_axis_name="s")

    scatter = pl.kernel(
        _scatter_body,
        mesh=mesh,
        out_type=[
            jax.ShapeDtypeStruct((NP, D), jnp.float32),
            jax.ShapeDtypeStruct((NP, D), jnp.float32),
        ],
        scratch_types=(
            [pltpu.VMEM((EPW,), jnp.int32)]
            + [pltpu.VMEM((K2,), jnp.float32) for _ in range(2)]
            + [pltpu.VMEM((K2, D), jnp.float32) for _ in range(2)]
            + [pltpu.VMEM((K2,), jnp.int32) for _ in range(2)]
            + [pltpu.VMEM_SHARED((NP, D), jnp.float32)]
            + [pltpu.SemaphoreType.DMA for _ in range(4)]
        ),
    )
    p0, p1 = scatter(src2, dst2, ew2, W)

    gather = pl.kernel(
        _gather_body,
        mesh=mesh,
        out_type=jax.ShapeDtypeStruct((B, D), jnp.float32),
        scratch_types=(
            [pltpu.VMEM((BPW,), jnp.int32)]
            + [pltpu.VMEM((KB, D), jnp.float32) for _ in range(NBCHUNK)]
            + [pltpu.SemaphoreType.DMA for _ in range(3 * NBCHUNK)]
        ),
    )
    return gather(p0, p1, x)


# speedup vs baseline: 3.1106x; 1.0109x over previous
import jax
import jax.numpy as jnp
from jax import lax
from jax.experimental import pallas as pl
from jax.experimental.pallas import tpu as pltpu
from jax.experimental.pallas import tpu_sc as plsc

N = 10000
E = 320000
D = 128
B = 16384

NC = 2
NS = 16
NW = NC * NS
L = 16

NP = 10112
RPT = NP // NS
K2 = 112
EPP = 180 * K2
NCH0 = 130
NCH1 = 50
EPW = NCH0 * K2
E2 = EPP * NS
BPW = B // NW
KB = 128
NBCHUNK = BPW // KB


def _lane_splat(vec, r):
    idx = jnp.full((L, 1), 0, jnp.int32) + r
    dn = lax.GatherDimensionNumbers(
        offset_dims=(), collapsed_slice_dims=(0,), start_index_map=(0,))
    return lax.gather(vec, idx, dn, (1,),
                      mode=lax.GatherScatterMode.PROMISE_IN_BOUNDS)


def _scale_chunk(rows_ref, ew_ref, ebase):
    def grp(g, carry):
        ew_vec = ew_ref[pl.ds(ebase + g * L, L)]
        for r in range(L):
            sv = _lane_splat(ew_vec, r)
            for j in range(D // L):
                rows_ref[g * L + r, pl.ds(j * L, L)] = (
                    rows_ref[g * L + r, pl.ds(j * L, L)] * sv)
        return carry
    lax.fori_loop(0, K2 // L, grp, 0)


def _scatter_body(src_h, dst_h, ew_h, w_h, p0_h, p1_h,
                  srcv, e0, e1, r0, r1, t0, t1, acc,
                  sg0, sg1, sd0, sd1):
    rows = [r0, r1]
    dss = [t0, t1]
    ewr = [e0, e1]
    sg = [sg0, sg1]
    sd = [sd0, sd1]

    c = lax.axis_index("c")
    s = lax.axis_index("s")
    ebase = s * EPP + c * (NCH0 * K2)
    nch = NCH0 - (NCH0 - NCH1) * c
    myepw = nch * K2

    def zrow(i, carry):
        for j in range(D // L):
            r0[i, pl.ds(j * L, L)] = jnp.zeros((L,), jnp.float32)
        return carry
    lax.fori_loop(0, K2, zrow, 0)
    for t in range(5):
        pltpu.sync_copy(r0, acc.at[pl.ds(s * RPT + t * K2, K2)])
    pltpu.sync_copy(r0.at[pl.ds(0, RPT - 5 * K2)],
                    acc.at[pl.ds(s * RPT + 5 * K2, RPT - 5 * K2)])

    pltpu.sync_copy(src_h.at[pl.ds(ebase, EPW)], srcv)
    plsc.subcore_barrier()

    def fetch_dst(i, b):
        pltpu.async_copy(dst_h.at[pl.ds(ebase + i * K2, K2)], dss[b], sd[b])
        pltpu.async_copy(ew_h.at[pl.ds(ebase + i * K2, K2)], ewr[b], sd[b])

    def start_gather(i, b):
        pltpu.async_copy(w_h.at[srcv.at[pl.ds(i * K2, K2)]], rows[b], sg[b])

    for b in range(2):
        fetch_dst(b, b)
        start_gather(b, b)

    def outer(p, carry):
        for b in range(2):
            i = 2 * p + b
            pltpu.make_async_copy(
                w_h.at[srcv.at[pl.ds(0, K2)]], rows[b], sg[b]).wait()
            pltpu.make_async_copy(
                dst_h.at[pl.ds(0, K2)], dss[b], sd[b]).wait()
            pltpu.make_async_copy(
                ew_h.at[pl.ds(0, K2)], ewr[b], sd[b]).wait()
            _scale_chunk(rows[b], ewr[b], 0)
            pltpu.sync_copy(rows[b], acc.at[dss[b]], add=True)
            @pl.when(i + 2 < nch)
            def _():
                fetch_dst(i + 2, b)
                start_gather(i + 2, b)
        return carry
    lax.fori_loop(0, nch // 2, outer, 0)

    plsc.subcore_barrier()

    @pl.when(c == 0)
    def _():
        pltpu.sync_copy(acc.at[pl.ds(s * RPT, RPT)], p0_h.at[pl.ds(s * RPT, RPT)])
    @pl.when(c == 1)
    def _():
        pltpu.sync_copy(acc.at[pl.ds(s * RPT, RPT)], p1_h.at[pl.ds(s * RPT, RPT)])


def _gather_body(p0_h, p1_h, x_h, out_h, xv,
                 r0, r1, r2, r3, g0, g1, g2, g3,
                 a0, a1, a2, a3, t0, t1, t2, t3):
    rows = [r0, r1, r2, r3]
    sgm = [g0, g1, g2, g3]
    sam = [a0, a1, a2, a3]
    stm = [t0, t1, t2, t3]
    c = lax.axis_index("c")
    s = lax.axis_index("s")
    wid = s * NC + c
    base = wid * BPW
    pltpu.sync_copy(x_h.at[pl.ds(base, BPW)], xv)
    for t in range(NBCHUNK):
        pltpu.async_copy(p0_h.at[xv.at[pl.ds(t * KB, KB)]], rows[t], sgm[t])
    for t in range(NBCHUNK):
        pltpu.make_async_copy(
            p0_h.at[xv.at[pl.ds(t * KB, KB)]], rows[t], sgm[t]).wait()
        pltpu.async_copy(p1_h.at[xv.at[pl.ds(t * KB, KB)]], rows[t],
                         sam[t], add=True)
        pltpu.make_async_copy(
            p1_h.at[xv.at[pl.ds(t * KB, KB)]], rows[t], sam[t]).wait()
        pltpu.async_copy(rows[t], out_h.at[pl.ds(base + t * KB, KB)], stm[t])
    for t in range(NBCHUNK):
        pltpu.make_async_copy(
            rows[t], out_h.at[pl.ds(base + t * KB, KB)], stm[t]).wait()


def kernel(x, edge_index, edge_weight, W):
    src = edge_index[0]
    dst = edge_index[1]
    pad = E2 - E
    src2 = jnp.concatenate([src, jnp.zeros((pad,), src.dtype)])
    dst2 = jnp.concatenate([dst, jnp.zeros((pad,), dst.dtype)])
    ew2 = jnp.concatenate([edge_weight, jnp.zeros((pad,), edge_weight.dtype)])

    mesh = plsc.VectorSubcoreMesh(core_axis_name="c", subcore_axis_name="s")

    scatter = pl.kernel(
        _scatter_body,
        mesh=mesh,
        out_type=[
            jax.ShapeDtypeStruct((NP, D), jnp.float32),
            jax.ShapeDtypeStruct((NP, D), jnp.float32),
        ],
        scratch_types=(
            [pltpu.VMEM((EPW,), jnp.int32)]
            + [pltpu.VMEM((K2,), jnp.float32) for _ in range(2)]
            + [pltpu.VMEM((K2, D), jnp.float32) for _ in range(2)]
            + [pltpu.VMEM((K2,), jnp.int32) for _ in range(2)]
            + [pltpu.VMEM_SHARED((NP, D), jnp.float32)]
            + [pltpu.SemaphoreType.DMA for _ in range(4)]
        ),
    )
    p0, p1 = scatter(src2, dst2, ew2, W)

    gather = pl.kernel(
        _gather_body,
        mesh=mesh,
        out_type=jax.ShapeDtypeStruct((B, D), jnp.float32),
        scratch_types=(
            [pltpu.VMEM((BPW,), jnp.int32)]
            + [pltpu.VMEM((KB, D), jnp.float32) for _ in range(NBCHUNK)]
            + [pltpu.SemaphoreType.DMA for _ in range(3 * NBCHUNK)]
        ),
    )
    return gather(p0, p1, x)


# speedup vs baseline: 3.1816x; 1.0228x over previous
import jax
import jax.numpy as jnp
from jax import lax
from jax.experimental import pallas as pl
from jax.experimental.pallas import tpu as pltpu
from jax.experimental.pallas import tpu_sc as plsc

N = 10000
E = 320000
D = 128
B = 16384

NC = 2
NS = 16
NW = NC * NS
L = 16

NP = 10112
RPT = NP // NS
K2 = 112
EPP = 180 * K2
NCH0 = 136
NCH1 = 44
EPW = NCH0 * K2
E2 = EPP * NS
BPW = B // NW
KB = 128
NBCHUNK = BPW // KB


def _lane_splat(vec, r):
    idx = jnp.full((L, 1), 0, jnp.int32) + r
    dn = lax.GatherDimensionNumbers(
        offset_dims=(), collapsed_slice_dims=(0,), start_index_map=(0,))
    return lax.gather(vec, idx, dn, (1,),
                      mode=lax.GatherScatterMode.PROMISE_IN_BOUNDS)


def _scale_chunk(rows_ref, ew_ref, ebase):
    def grp(g, carry):
        ew_vec = ew_ref[pl.ds(ebase + g * L, L)]
        for r in range(L):
            sv = _lane_splat(ew_vec, r)
            for j in range(D // L):
                rows_ref[g * L + r, pl.ds(j * L, L)] = (
                    rows_ref[g * L + r, pl.ds(j * L, L)] * sv)
        return carry
    lax.fori_loop(0, K2 // L, grp, 0)


def _scatter_body(src_h, dst_h, ew_h, w_h, p0_h, p1_h,
                  srcv, e0, e1, r0, r1, t0, t1, acc,
                  sg0, sg1, sd0, sd1):
    rows = [r0, r1]
    dss = [t0, t1]
    ewr = [e0, e1]
    sg = [sg0, sg1]
    sd = [sd0, sd1]

    c = lax.axis_index("c")
    s = lax.axis_index("s")
    ebase = s * EPP + c * (NCH0 * K2)
    nch = NCH0 - (NCH0 - NCH1) * c
    myepw = nch * K2

    def zrow(i, carry):
        for j in range(D // L):
            r0[i, pl.ds(j * L, L)] = jnp.zeros((L,), jnp.float32)
        return carry
    lax.fori_loop(0, K2, zrow, 0)
    for t in range(5):
        pltpu.sync_copy(r0, acc.at[pl.ds(s * RPT + t * K2, K2)])
    pltpu.sync_copy(r0.at[pl.ds(0, RPT - 5 * K2)],
                    acc.at[pl.ds(s * RPT + 5 * K2, RPT - 5 * K2)])

    pltpu.sync_copy(src_h.at[pl.ds(ebase, EPW)], srcv)
    plsc.subcore_barrier()

    def fetch_dst(i, b):
        pltpu.async_copy(dst_h.at[pl.ds(ebase + i * K2, K2)], dss[b], sd[b])
        pltpu.async_copy(ew_h.at[pl.ds(ebase + i * K2, K2)], ewr[b], sd[b])

    def start_gather(i, b):
        pltpu.async_copy(w_h.at[srcv.at[pl.ds(i * K2, K2)]], rows[b], sg[b])

    for b in range(2):
        fetch_dst(b, b)
        start_gather(b, b)

    def outer(p, carry):
        for b in range(2):
            i = 2 * p + b
            pltpu.make_async_copy(
                w_h.at[srcv.at[pl.ds(0, K2)]], rows[b], sg[b]).wait()
            pltpu.make_async_copy(
                dst_h.at[pl.ds(0, K2)], dss[b], sd[b]).wait()
            pltpu.make_async_copy(
                ew_h.at[pl.ds(0, K2)], ewr[b], sd[b]).wait()
            _scale_chunk(rows[b], ewr[b], 0)
            pltpu.sync_copy(rows[b], acc.at[dss[b]], add=True)
            @pl.when(i + 2 < nch)
            def _():
                fetch_dst(i + 2, b)
                start_gather(i + 2, b)
        return carry
    lax.fori_loop(0, nch // 2, outer, 0)

    plsc.subcore_barrier()

    @pl.when(c == 0)
    def _():
        pltpu.sync_copy(acc.at[pl.ds(s * RPT, RPT)], p0_h.at[pl.ds(s * RPT, RPT)])
    @pl.when(c == 1)
    def _():
        pltpu.sync_copy(acc.at[pl.ds(s * RPT, RPT)], p1_h.at[pl.ds(s * RPT, RPT)])


def _gather_body(p0_h, p1_h, x_h, out_h, xv,
                 r0, r1, r2, r3, g0, g1, g2, g3,
                 a0, a1, a2, a3, t0, t1, t2, t3):
    rows = [r0, r1, r2, r3]
    sgm = [g0, g1, g2, g3]
    sam = [a0, a1, a2, a3]
    stm = [t0, t1, t2, t3]
    c = lax.axis_index("c")
    s = lax.axis_index("s")
    wid = s * NC + c
    base = wid * BPW
    pltpu.sync_copy(x_h.at[pl.ds(base, BPW)], xv)
    for t in range(NBCHUNK):
        pltpu.async_copy(p0_h.at[xv.at[pl.ds(t * KB, KB)]], rows[t], sgm[t])
    for t in range(NBCHUNK):
        pltpu.make_async_copy(
            p0_h.at[xv.at[pl.ds(t * KB, KB)]], rows[t], sgm[t]).wait()
        pltpu.async_copy(p1_h.at[xv.at[pl.ds(t * KB, KB)]], rows[t],
                         sam[t], add=True)
        pltpu.make_async_copy(
            p1_h.at[xv.at[pl.ds(t * KB, KB)]], rows[t], sam[t]).wait()
        pltpu.async_copy(rows[t], out_h.at[pl.ds(base + t * KB, KB)], stm[t])
    for t in range(NBCHUNK):
        pltpu.make_async_copy(
            rows[t], out_h.at[pl.ds(base + t * KB, KB)], stm[t]).wait()


def kernel(x, edge_index, edge_weight, W):
    src = edge_index[0]
    dst = edge_index[1]
    pad = E2 - E
    src2 = jnp.concatenate([src, jnp.zeros((pad,), src.dtype)])
    dst2 = jnp.concatenate([dst, jnp.zeros((pad,), dst.dtype)])
    ew2 = jnp.concatenate([edge_weight, jnp.zeros((pad,), edge_weight.dtype)])

    mesh = plsc.VectorSubcoreMesh(core_axis_name="c", subcore_axis_name="s")

    scatter = pl.kernel(
        _scatter_body,
        mesh=mesh,
        out_type=[
            jax.ShapeDtypeStruct((NP, D), jnp.float32),
            jax.ShapeDtypeStruct((NP, D), jnp.float32),
        ],
        scratch_types=(
            [pltpu.VMEM((EPW,), jnp.int32)]
            + [pltpu.VMEM((K2,), jnp.float32) for _ in range(2)]
            + [pltpu.VMEM((K2, D), jnp.float32) for _ in range(2)]
            + [pltpu.VMEM((K2,), jnp.int32) for _ in range(2)]
            + [pltpu.VMEM_SHARED((NP, D), jnp.float32)]
            + [pltpu.SemaphoreType.DMA for _ in range(4)]
        ),
    )
    p0, p1 = scatter(src2, dst2, ew2, W)

    gather = pl.kernel(
        _gather_body,
        mesh=mesh,
        out_type=jax.ShapeDtypeStruct((B, D), jnp.float32),
        scratch_types=(
            [pltpu.VMEM((BPW,), jnp.int32)]
            + [pltpu.VMEM((KB, D), jnp.float32) for _ in range(NBCHUNK)]
            + [pltpu.SemaphoreType.DMA for _ in range(3 * NBCHUNK)]
        ),
    )
    return gather(p0, p1, x)
